# baseline (speedup 1.0000x reference)
import functools

import jax
import jax.numpy as jnp
from jax import lax
from jax.experimental import pallas as pl
from jax.experimental.pallas import tpu as pltpu
from jax.experimental.pallas import tpu_sc as plsc

B = 16384
D = 128
S = 8
P = 16
L = 16
NC = 2
NS = 16
NW = NC * NS
BPW = B // NW
G = 1


def _sc_body(feat_hbm, stages_hbm, protos_hbm, out_hbm,
             protos_v, means_v, stg_v, feat_v, pr_sem, st_sem, in_sem):
    wid = lax.axis_index("s") * NC + lax.axis_index("c")
    base = wid * BPW

    pr_copy = pltpu.async_copy(protos_hbm, protos_v, pr_sem)
    st_copy = pltpu.async_copy(stages_hbm.at[pl.ds(base, BPW)],
                               stg_v.at[pl.ds(0, BPW)], st_sem)
    in_copy = pltpu.async_copy(feat_hbm.at[pl.ds(base, BPW)], feat_v, in_sem)
    pr_copy.wait()

    scale = 0.5 / P

    @plsc.parallel_loop(0, S * (D // L))
    def _(sj):
        s = sj // (D // L)
        j = sj % (D // L)
        acc = protos_v[s, 0, pl.ds(j * L, L)]
        for p in range(1, P):
            acc = acc + protos_v[s, p, pl.ds(j * L, L)]
        means_v[pl.ds(s * D + j * L, L)] = acc * scale

    st_copy.wait()
    in_copy.wait()

    @plsc.parallel_loop(0, BPW // G)
    def _(g):
        stv = stg_v[pl.ds(g * G, L)]
        for k in range(G):
            i = g * G + k
            st_off = stv[k] * D
            for j in range(D // L):
                m = means_v[pl.ds(st_off + j * L, L)]
                f = feat_v[i, pl.ds(j * L, L)]
                feat_v[i, pl.ds(j * L, L)] = f + m

    pltpu.sync_copy(feat_v, out_hbm.at[pl.ds(base, BPW)])


def kernel(features, class_ids, stages, shared_protos):
    del class_ids
    stages_i32 = stages.astype(jnp.int32)
    mesh = plsc.VectorSubcoreMesh(core_axis_name="c", subcore_axis_name="s")
    k = functools.partial(
        pl.kernel,
        mesh=mesh,
        out_type=jax.ShapeDtypeStruct((B, D), jnp.float32),
        scratch_types=[
            pltpu.VMEM((S, P, D), jnp.float32),
            pltpu.VMEM((S * D,), jnp.float32),
            pltpu.VMEM((BPW + L,), jnp.int32),
            pltpu.VMEM((BPW, D), jnp.float32),
            pltpu.SemaphoreType.DMA,
            pltpu.SemaphoreType.DMA,
            pltpu.SemaphoreType.DMA,
        ],
    )(_sc_body)
    return k(features, stages_i32, shared_protos)


# speedup vs baseline: 1.0175x; 1.0175x over previous
import functools

import jax
import jax.numpy as jnp
from jax import lax
from jax.experimental import pallas as pl
from jax.experimental.pallas import tpu as pltpu
from jax.experimental.pallas import tpu_sc as plsc

B = 16384
D = 128
S = 8
P = 16
L = 16
NC = 2
NS = 16
NW = NC * NS
BPW = B // NW
G = 2
NCHUNK = 2
CS = BPW // NCHUNK


def _sc_body(feat_hbm, stages_hbm, protos_hbm, out_hbm,
             protos_v, means_v, stg_v, feat_v,
             pr_sem, st_sem, in_sem0, in_sem1, out_sem0, out_sem1):
    wid = lax.axis_index("s") * NC + lax.axis_index("c")
    base = wid * BPW
    in_sems = (in_sem0, in_sem1)
    out_sems = (out_sem0, out_sem1)

    pr_copy = pltpu.async_copy(protos_hbm, protos_v, pr_sem)
    st_copy = pltpu.async_copy(stages_hbm.at[pl.ds(base, BPW)],
                               stg_v.at[pl.ds(0, BPW)], st_sem)
    in_copies = [
        pltpu.async_copy(feat_hbm.at[pl.ds(base + c * CS, CS)],
                         feat_v.at[pl.ds(c * CS, CS)], in_sems[c])
        for c in range(NCHUNK)
    ]
    pr_copy.wait()

    scale = 0.5 / P

    @plsc.parallel_loop(0, S * (D // L))
    def _(sj):
        s = sj // (D // L)
        j = sj % (D // L)
        acc = protos_v[s, 0, pl.ds(j * L, L)]
        for p in range(1, P):
            acc = acc + protos_v[s, p, pl.ds(j * L, L)]
        means_v[pl.ds(s * D + j * L, L)] = acc * scale

    st_copy.wait()

    out_copies = [None] * NCHUNK
    for c in range(NCHUNK):
        in_copies[c].wait()

        @plsc.parallel_loop(0, CS // G)
        def _(g):
            i0 = c * CS + g * G
            stv = stg_v[pl.ds(i0, L)]
            for k in range(G):
                i = i0 + k
                st_off = stv[k] * D
                for j in range(D // L):
                    m = means_v[pl.ds(st_off + j * L, L)]
                    f = feat_v[i, pl.ds(j * L, L)]
                    feat_v[i, pl.ds(j * L, L)] = f + m

        out_copies[c] = pltpu.async_copy(
            feat_v.at[pl.ds(c * CS, CS)],
            out_hbm.at[pl.ds(base + c * CS, CS)], out_sems[c])

    for c in range(NCHUNK):
        out_copies[c].wait()


def kernel(features, class_ids, stages, shared_protos):
    del class_ids
    stages_i32 = stages.astype(jnp.int32)
    mesh = plsc.VectorSubcoreMesh(core_axis_name="c", subcore_axis_name="s")
    k = functools.partial(
        pl.kernel,
        mesh=mesh,
        out_type=jax.ShapeDtypeStruct((B, D), jnp.float32),
        scratch_types=[
            pltpu.VMEM((S, P, D), jnp.float32),
            pltpu.VMEM((S * D,), jnp.float32),
            pltpu.VMEM((BPW + L,), jnp.int32),
            pltpu.VMEM((BPW, D), jnp.float32),
            pltpu.SemaphoreType.DMA,
            pltpu.SemaphoreType.DMA,
            pltpu.SemaphoreType.DMA,
            pltpu.SemaphoreType.DMA,
            pltpu.SemaphoreType.DMA,
            pltpu.SemaphoreType.DMA,
        ],
    )(_sc_body)
    return k(features, stages_i32, shared_protos)
